# baseline (speedup 1.0000x reference)
import functools

import jax
import jax.numpy as jnp
from jax import lax
from jax.experimental import pallas as pl
from jax.experimental.pallas import tpu as pltpu
from jax.experimental.pallas import tpu_sc as plsc

_TOPK = 8
_E = 64
_D = 4096
_NW = 32
_NCHUNK = 4


def _logits_block(x_ref, w_ref, out_ref):
    out_ref[...] = jax.lax.dot_general(
        x_ref[...], w_ref[...], (((1,), (1,)), ((), ())),
        preferred_element_type=jnp.float32,
    )


def _tc_logits_chunk(x, W, chunk, chunk_rows, block):
    base = chunk * (chunk_rows // block)
    return pl.pallas_call(
        _logits_block,
        grid=(chunk_rows // block,),
        in_specs=[
            pl.BlockSpec((block, _D), lambda i: (base + i, 0)),
            pl.BlockSpec((_E, _D), lambda i: (0, 0)),
        ],
        out_specs=pl.BlockSpec((block, _E), lambda i: (i, 0)),
        out_shape=jax.ShapeDtypeStruct((chunk_rows, _E), jnp.float32),
    )(x, W)


def _sc_router_body(logits_hbm, scores_hbm, ew_hbm, ei_hbm,
                    lg_v, sc_v, ew_v, ei_v, sem):
    rows = lg_v.shape[0]
    wid = lax.axis_index("s") * 2 + lax.axis_index("c")
    base = wid * rows
    pltpu.sync_copy(logits_hbm.at[pl.ds(base, rows)], lg_v)

    lane = lax.iota(jnp.int32, 16)
    lane_u = lane.astype(jnp.uint32)
    lane_lt8 = lane < _TOPK

    def row_body(r, carry):
        s = [lg_v[r, pl.ds(16 * c, 16)] for c in range(4)]
        m = jnp.maximum(jnp.maximum(s[0], s[1]), jnp.maximum(s[2], s[3]))
        mx = jnp.max(m)
        e = [jnp.exp(v - mx) for v in s]
        denom = jnp.sum(e[0] + e[1] + e[2] + e[3])
        inv = jnp.full((16,), 1.0, jnp.float32) / jnp.broadcast_to(denom, (16,))
        p = [v * inv for v in e]
        keys = []
        for c in range(4):
            sc_v[r, pl.ds(16 * c, 16)] = p[c]
            bits = plsc.bitcast(p[c], jnp.uint32)
            rounded = ((bits << jnp.uint32(2)) + jnp.uint32(8)) & jnp.uint32(
                0xFFFFFFC0
            )
            keys.append(rounded | (jnp.uint32(63 - 16 * c) - lane_u))
        ks = [plsc.sort_key_val(k, k, descending=True)[0] for k in keys]
        m01 = jnp.maximum(ks[0], lax.rev(ks[1], (0,)))
        m23 = jnp.maximum(ks[2], lax.rev(ks[3], (0,)))
        t01 = plsc.sort_key_val(m01, m01, descending=True)[0]
        t23 = plsc.sort_key_val(m23, m23, descending=True)[0]
        mt = jnp.maximum(t01, lax.rev(t23, (0,)))
        top = plsc.sort_key_val(mt, mt, descending=True)[0]
        idx8 = (jnp.uint32(63) - (top & jnp.uint32(63))).astype(jnp.int32)
        rvec = jnp.full((16,), r, jnp.int32)
        w16 = plsc.load_gather(sc_v, [rvec, idx8])
        wsel = jnp.where(lane_lt8, w16, 0.0)
        wnorm = wsel / jnp.broadcast_to(jnp.sum(wsel), (16,))
        plsc.store_scatter(ew_v, [rvec, lane], wnorm, mask=lane_lt8)
        plsc.store_scatter(ei_v, [rvec, lane], idx8, mask=lane_lt8)
        return carry

    lax.fori_loop(0, rows, row_body, 0)

    pltpu.sync_copy(sc_v, scores_hbm.at[pl.ds(base, rows)])
    pltpu.sync_copy(ew_v, ew_hbm.at[pl.ds(base, rows)])
    pltpu.sync_copy(ei_v, ei_hbm.at[pl.ds(base, rows)])


def _sc_router(logits):
    n_tokens = logits.shape[0]
    rows = n_tokens // _NW
    mesh = plsc.VectorSubcoreMesh(core_axis_name="c", subcore_axis_name="s")
    f = functools.partial(
        pl.kernel,
        mesh=mesh,
        out_type=[
            jax.ShapeDtypeStruct((n_tokens, _E), jnp.float32),
            jax.ShapeDtypeStruct((n_tokens, _TOPK), jnp.float32),
            jax.ShapeDtypeStruct((n_tokens, _TOPK), jnp.int32),
        ],
        scratch_types=[
            pltpu.VMEM((rows, _E), jnp.float32),
            pltpu.VMEM((rows, _E), jnp.float32),
            pltpu.VMEM((rows, _TOPK), jnp.float32),
            pltpu.VMEM((rows, _TOPK), jnp.int32),
            pltpu.SemaphoreType.DMA,
        ],
        compiler_params=pltpu.CompilerParams(needs_layout_passes=False),
    )(_sc_router_body)
    return f(logits)


def kernel(x, W):
    n_tokens = x.shape[0]
    chunk_rows = n_tokens // _NCHUNK
    parts = []
    for c in range(_NCHUNK):
        logits_c = _tc_logits_chunk(x, W, c, chunk_rows, block=1024)
        parts.append(_sc_router(logits_c))
    scores = jnp.concatenate([p[0] for p in parts], axis=0)
    ew = jnp.concatenate([p[1] for p in parts], axis=0)
    ei = jnp.concatenate([p[2] for p in parts], axis=0)
    return (scores, ew, ei)


# speedup vs baseline: 1.4486x; 1.4486x over previous
import jax
import jax.numpy as jnp
from jax.experimental import pallas as pl

_TOPK = 8
_E = 64
_D = 4096


def _router_block(x_ref, w_ref, scores_ref, ew_ref, ei_ref):
    x = x_ref[...]
    w = w_ref[...]
    logits = jax.lax.dot_general(
        x, w, (((1,), (1,)), ((), ())), preferred_element_type=jnp.float32
    )
    mx = jnp.max(logits, axis=1, keepdims=True)
    e = jnp.exp(logits - mx)
    denom = jnp.sum(e, axis=1, keepdims=True)
    scores = e / denom
    scores_ref[...] = scores

    iota = jax.lax.broadcasted_iota(jnp.int32, scores.shape, 1)
    s = scores
    vals, idxs = [], []
    for _ in range(_TOPK):
        m = jnp.max(s, axis=1, keepdims=True)
        idx = jnp.min(jnp.where(s == m, iota, _E), axis=1, keepdims=True)
        vals.append(m)
        idxs.append(idx)
        s = jnp.where(iota == idx, -1.0, s)
    w8 = jnp.concatenate(vals, axis=1)
    i8 = jnp.concatenate(idxs, axis=1)
    ew_ref[...] = w8 / jnp.sum(w8, axis=1, keepdims=True)
    ei_ref[...] = i8


def kernel(x, W):
    n_tokens = x.shape[0]
    block = 1024
    grid = (n_tokens // block,)
    scores, ew, ei = pl.pallas_call(
        _router_block,
        grid=grid,
        in_specs=[
            pl.BlockSpec((block, _D), lambda i: (i, 0)),
            pl.BlockSpec((_E, _D), lambda i: (0, 0)),
        ],
        out_specs=[
            pl.BlockSpec((block, _E), lambda i: (i, 0)),
            pl.BlockSpec((block, _TOPK), lambda i: (i, 0)),
            pl.BlockSpec((block, _TOPK), lambda i: (i, 0)),
        ],
        out_shape=[
            jax.ShapeDtypeStruct((n_tokens, _E), jnp.float32),
            jax.ShapeDtypeStruct((n_tokens, _TOPK), jnp.float32),
            jax.ShapeDtypeStruct((n_tokens, _TOPK), jnp.int32),
        ],
    )(x, W)
    return (scores, ew, ei)
